# Initial kernel scaffold; baseline (speedup 1.0000x reference)
import jax
import jax.numpy as jnp
from jax.experimental import pallas as pl


def kernel(xA_pos, xB_pos, s, is_bond_A, is_bond_B, params, Z, edge_index):
    raise NotImplementedError("write your pallas kernel here")


# speedup vs baseline: 1.0661x; 1.0661x over previous
import functools
import math

import jax
import jax.numpy as jnp
import numpy as np
from jax import lax
from jax.experimental import pallas as pl
from jax.experimental.pallas import tpu as pltpu

_N = 10000
_E = 320000
_N_FREQ = 8
_N_LAYERS = 3
_HID = 64
_D_CUTOFF = 5.0
_N_RBF = 10
_N_Z = 100
_STATE = 80

_NBLK = 2000
_EBLK = 4000

_CENTERS = np.linspace(0.0, _D_CUTOFF, _N_RBF).astype(np.float32)
_GAMMA = float(1.0 / (_CENTERS[1] - _CENTERS[0]) ** 2)


def _rbf(d):
    step = _D_CUTOFF / (_N_RBF - 1)
    c = lax.broadcasted_iota(jnp.int32, (1, _N_RBF), 1).astype(jnp.float32) * step
    return jnp.exp(-_GAMMA * (d - c) ** 2)


def _gelu(x):
    return jax.nn.gelu(x)


def _rowcall(fn, nrows, block, row_ins, full_ins, out_cols):
    grid = (nrows // block,)

    def mk_row_spec(a):
        nd = a.ndim
        return pl.BlockSpec((block,) + a.shape[1:],
                            lambda i, _nd=nd: (i,) + (0,) * (_nd - 1))

    def mk_full_spec(a):
        nd = a.ndim
        return pl.BlockSpec(a.shape, lambda i, _nd=nd: (0,) * _nd)

    in_specs = [mk_row_spec(a) for a in row_ins] + [mk_full_spec(a) for a in full_ins]
    out_shapes = [jax.ShapeDtypeStruct((nrows, c), jnp.float32) for c in out_cols]
    out_specs = [pl.BlockSpec((block, c), lambda i: (i, 0)) for c in out_cols]
    res = pl.pallas_call(
        fn,
        grid=grid,
        in_specs=in_specs,
        out_specs=out_specs,
        out_shape=out_shapes,
    )(*row_ins, *full_ins)
    return res


def _mm(a, w):
    return jnp.dot(a, w, preferred_element_type=jnp.float32)


def _node0_body(z_ref, s_ref, Wi0, bi0, Wi1, bi1, WA0, bA0, WB0, bB0,
                tA_ref, tB_ref, h0_ref, semb_ref):
    z = z_ref[...]
    onehot = (lax.broadcasted_iota(jnp.int32, (z.shape[0], _N_Z), 1) == z).astype(jnp.float32)
    ie = _gelu(_mm(onehot, Wi0[...]) + bi0[...])
    atom = _mm(ie, Wi1[...]) + bi1[...]
    tA = _gelu(_mm(onehot, WA0[...]) + bA0[...])
    tB = _gelu(_mm(onehot, WB0[...]) + bB0[...])
    sv = s_ref[...]
    freqs = lax.broadcasted_iota(jnp.int32, (1, _N_FREQ), 1).astype(jnp.float32) + 1.0
    ang = np.pi * sv * freqs
    semb = jnp.concatenate([jnp.sin(ang), jnp.cos(ang)], axis=1)
    tA_ref[...] = tA
    tB_ref[...] = tB
    h0_ref[...] = jnp.concatenate([atom, semb], axis=1)
    semb_ref[...] = semb


def _efstatic_body(ga_ref, gb_ref, bond_ref, ef_ref):
    d16 = ga_ref[...] - gb_ref[...]
    dxA = d16[:, 0:3]
    dxB = d16[:, 4:7]
    dA = jnp.sqrt(jnp.sum(dxA * dxA, axis=1, keepdims=True) + 1e-12)
    dB = jnp.sqrt(jnp.sum(dxB * dxB, axis=1, keepdims=True) + 1e-12)
    z = jnp.zeros((d16.shape[0], 7), jnp.float32)
    ef_ref[...] = jnp.concatenate(
        [bond_ref[...], dA, dB, dA - dB, _rbf(dA), _rbf(dB), z], axis=1)


def _node1_body(tA_ref, tB_ref, aggA_ref, aggB_ref, semb_ref, h0_ref,
                W1At, W1Ab, b1A, W1Bt, W1Bb, b1B,
                Wf1_0, bf1_0, Wf2_0, bf2_0,
                Wf1_1, bf1_1, Wf2_1, bf2_1,
                Wf1_2, bf1_2, Wf2_2, bf2_2,
                Wms0, Wmd0, bm0,
                f0_ref, f1_ref, f2_ref, um_ref):
    aggA = aggA_ref[:, :64] + aggA_ref[:, 64:]
    aggB = aggB_ref[:, :64] + aggB_ref[:, 64:]
    hA = _gelu(_mm(tA_ref[...], W1At[...]) + _mm(aggA, W1Ab[...]) + b1A[...])
    hB = _gelu(_mm(tB_ref[...], W1Bt[...]) + _mm(aggB, W1Bb[...]) + b1B[...])
    fi = jnp.concatenate([hA, hB, semb_ref[...]], axis=1)
    f0_ref[...] = _mm(_gelu(_mm(fi, Wf1_0[...]) + bf1_0[...]), Wf2_0[...]) + bf2_0[...]
    f1_ref[...] = _mm(_gelu(_mm(fi, Wf1_1[...]) + bf1_1[...]), Wf2_1[...]) + bf2_1[...]
    f2_ref[...] = _mm(_gelu(_mm(fi, Wf1_2[...]) + bf1_2[...]), Wf2_2[...]) + bf2_2[...]
    h0 = h0_ref[...]
    um_ref[...] = jnp.concatenate([_mm(h0, Wms0[...]), _mm(h0, Wmd0[...]) + bm0[...]], axis=1)


def _edge_dyn(dx16, ef_ref, We_s, We_d):
    dx = dx16[:, 0:3]
    dist = jnp.sqrt(jnp.sum(dx * dx, axis=1, keepdims=True) + 1e-12)
    efd = jnp.concatenate([dist, dist * dist, _rbf(dist)], axis=1)
    return _mm(ef_ref, We_s) + _mm(efd, We_d)


def _msg_body(gs_ref, gd_ref, xs_ref, xd_ref, ef_ref,
              We_s, We_d, Wm1, bm1, Wm2, bm2,
              msg_ref, dx_ref):
    dx16 = xs_ref[...] - xd_ref[...]
    pre = gs_ref[...] + gd_ref[...] + _edge_dyn(dx16, ef_ref[...], We_s[...], We_d[...])
    z = _gelu(pre)
    z = _gelu(_mm(z, Wm1[...]) + bm1[...])
    msg_ref[...] = _mm(z, Wm2[...]) + bm2[...]
    dx_ref[...] = dx16


def _state_body(h_ref, ms_ref, f_ref,
                Ws_h, Ws_m, bs1, Ws2, bs2, Ws3, bs3,
                Wa_s, Wa_d, ba,
                Wb1, bb1, Wb2, bb2, Wb3, bb3,
                Wg1, bg1, Wg2, bg2, Wg3, bg3,
                Wms, Wmd, bm,
                h_out, ua_ref, bg_ref, um_ref):
    h = h_ref[...]
    nm = ms_ref[:, :64] + ms_ref[:, 64:]
    u = _gelu(_mm(h, Ws_h[...]) + _mm(nm, Ws_m[...]) + bs1[...])
    u = _gelu(_mm(u, Ws2[...]) + bs2[...])
    h2 = h + _mm(u, Ws3[...]) + bs3[...]
    g = f_ref[:, :_STATE]
    b = f_ref[:, _STATE:]
    h2 = _gelu((1.0 + g) * h2 + b)
    h_out[...] = h2
    ua_ref[...] = jnp.concatenate([_mm(h2, Wa_s[...]), _mm(h2, Wa_d[...]) + ba[...]], axis=1)
    zb = _gelu(_mm(h2, Wb1[...]) + bb1[...])
    zb = _gelu(_mm(zb, Wb2[...]) + bb2[...])
    bet = _mm(zb, Wb3[...]) + bb3[...]
    zg = _gelu(_mm(h2, Wg1[...]) + bg1[...])
    zg = _gelu(_mm(zg, Wg2[...]) + bg2[...])
    gam = _mm(zg, Wg3[...]) + bg3[...]
    bg_ref[...] = jnp.concatenate([bet, gam], axis=1)
    um_ref[...] = jnp.concatenate([_mm(h2, Wms[...]), _mm(h2, Wmd[...]) + bm[...]], axis=1)


def _alpha_body(gs_ref, gd_ref, dx_ref, ef_ref,
                Wae_s, Wae_d, Wa1, ba1, Wa2, ba2,
                av_ref):
    dx16 = dx_ref[...]
    pre = gs_ref[...] + gd_ref[...] + _edge_dyn(dx16, ef_ref[...], Wae_s[...], Wae_d[...])
    z = _gelu(pre)
    z = _gelu(_mm(z, Wa1[...]) + ba1[...])
    alpha = _mm(z, Wa2[...]) + ba2[...]
    av_ref[...] = alpha * dx16


def _xup_body(x_ref, nu_ref, bg_ref, s_ref, xA_ref, xB_ref, xo_ref):
    x = x_ref[...]
    nu = nu_ref[:, :16] + nu_ref[:, 16:]
    bet = bg_ref[:, 0:1]
    gam = bg_ref[:, 1:2]
    sv = s_ref[...]
    xo_ref[...] = (x + nu + bet * (1.0 - sv) * (xA_ref[...] - x)
                   + gam * sv * (xB_ref[...] - x))


def _final_body(x_ref, xA_ref, xB_ref, s_ref, out_ref):
    sv = s_ref[...]
    base = (1.0 - sv) * xA_ref[...] + sv * xB_ref[...]
    corr = x_ref[...] - base
    xf = base + sv * (1.0 - sv) * corr
    out_ref[...] = xf - jnp.sum(xf, axis=0, keepdims=True) * (1.0 / _N)


def _gather_rows(table, idx):
    return jnp.take(table, idx, axis=0)


def _scatter_add2(data, idx, n, w):
    seg = jax.ops.segment_sum(data, idx, num_segments=n)
    return jnp.concatenate([seg, jnp.zeros((n, w), jnp.float32)], axis=1)


def _pack_edge_first(W, b):
    Ws = W[:_STATE]
    Wd = W[_STATE:2 * _STATE]
    We = W[2 * _STATE:]
    stat = jnp.concatenate([We[0][None], We[1][None], We[4][None], We[5][None],
                            We[6][None], We[17:27], We[27:37],
                            jnp.zeros((7, We.shape[1]), jnp.float32)], axis=0)
    dyn = jnp.concatenate([We[2][None], We[3][None], We[7:17]], axis=0)
    return Ws, Wd, stat, dyn, b[None, :]


def _r2(b):
    return b[None, :]


def kernel(xA_pos, xB_pos, s, is_bond_A, is_bond_B, params, Z, edge_index):
    s2 = s.reshape(_N, 1)
    z2 = Z.reshape(_N, 1).astype(jnp.int32)
    src = edge_index[0].astype(jnp.int32)
    dst = edge_index[1].astype(jnp.int32)
    pad13 = jnp.zeros((_N, 13), jnp.float32)
    xA16 = jnp.concatenate([xA_pos, pad13], axis=1)
    xB16 = jnp.concatenate([xB_pos, pad13], axis=1)
    xAB16 = jnp.concatenate([xA_pos, jnp.zeros((_N, 1), jnp.float32),
                             xB_pos, jnp.zeros((_N, 9), jnp.float32)], axis=1)
    bond2 = jnp.stack([is_bond_A, is_bond_B], axis=1)

    p = params
    Wi0, bi0 = p['info'][0]
    Wi1, bi1 = p['info'][1]
    WA0, bA0 = p['embA'][0]
    WB0, bB0 = p['embB'][0]
    W1A, b1A = p['embA'][1]
    W1B, b1B = p['embB'][1]

    tA, tB, h0, semb = _rowcall(
        _node0_body, _N, _NBLK, [z2, s2],
        [Wi0, _r2(bi0), Wi1, _r2(bi1), WA0, _r2(bA0), WB0, _r2(bB0)],
        [64, 64, 80, 16])

    gA = _gather_rows(xAB16, src)
    gB = _gather_rows(xAB16, dst)
    ef = _rowcall(_efstatic_body, _E, _EBLK, [gA, gB, bond2], [], [32])[0]

    aggA = _scatter_add2(_gather_rows(tA, src), dst, _N, 64)
    aggB = _scatter_add2(_gather_rows(tB, src), dst, _N, 64)

    lw = p['layers']
    film_w = []
    for l in range(_N_LAYERS):
        Wf1, bf1 = lw[l]['film'][0]
        Wf2, bf2 = lw[l]['film'][1]
        film_w += [Wf1, _r2(bf1), Wf2, _r2(bf2)]
    msg_first = [_pack_edge_first(*lw[l]['msg'][0]) for l in range(_N_LAYERS)]
    alpha_first = [_pack_edge_first(*lw[l]['alpha'][0]) for l in range(_N_LAYERS)]

    f0, f1, f2, um = _rowcall(
        _node1_body, _N, _NBLK, [tA, tB, aggA, aggB, semb, h0],
        [W1A[:64], W1A[64:], _r2(b1A), W1B[:64], W1B[64:], _r2(b1B)]
        + film_w
        + [msg_first[0][0], msg_first[0][1], msg_first[0][4]],
        [160, 160, 160, 128])
    films = [f0, f1, f2]

    h = h0
    x16 = _rowcall(
        lambda s_ref, xA_ref, xB_ref, o_ref: o_ref.__setitem__(
            ..., (1.0 - s_ref[...]) * xA_ref[...] + s_ref[...] * xB_ref[...]),
        _N, _NBLK, [s2, xA16, xB16], [], [16])[0]

    for l in range(_N_LAYERS):
        lp = lw[l]
        _, _, We_s, We_d, _ = msg_first[l]
        Wa_s, Wa_d, Wae_s, Wae_d, ba = alpha_first[l]

        gs = _gather_rows(um[:, :64], src)
        gd = _gather_rows(um[:, 64:], dst)
        xs = _gather_rows(x16, src)
        xd = _gather_rows(x16, dst)

        msg, dx16 = _rowcall(
            _msg_body, _E, _EBLK, [gs, gd, xs, xd, ef],
            [We_s, We_d, lp['msg'][1][0], _r2(lp['msg'][1][1]),
             lp['msg'][2][0], _r2(lp['msg'][2][1])],
            [64, 16])

        mslab = _scatter_add2(msg, dst, _N, 64)

        Ws0, bs0 = lp['state'][0]
        next_l = min(l + 1, _N_LAYERS - 1)
        Wms_n, Wmd_n, _, _, bm_n = msg_first[next_l]
        h, ua, bgv, um = _rowcall(
            _state_body, _N, _NBLK, [h, mslab, films[l]],
            [Ws0[:_STATE], Ws0[_STATE:], _r2(bs0),
             lp['state'][1][0], _r2(lp['state'][1][1]),
             lp['state'][2][0], _r2(lp['state'][2][1]),
             Wa_s, Wa_d, ba,
             lp['beta'][0][0], _r2(lp['beta'][0][1]),
             lp['beta'][1][0], _r2(lp['beta'][1][1]),
             lp['beta'][2][0], _r2(lp['beta'][2][1]),
             lp['gamma'][0][0], _r2(lp['gamma'][0][1]),
             lp['gamma'][1][0], _r2(lp['gamma'][1][1]),
             lp['gamma'][2][0], _r2(lp['gamma'][2][1]),
             Wms_n, Wmd_n, bm_n],
            [80, 128, 2, 128])

        gas = _gather_rows(ua[:, :64], src)
        gad = _gather_rows(ua[:, 64:], dst)
        avdx = _rowcall(
            _alpha_body, _E, _EBLK, [gas, gad, dx16, ef],
            [Wae_s, Wae_d, lp['alpha'][1][0], _r2(lp['alpha'][1][1]),
             lp['alpha'][2][0], _r2(lp['alpha'][2][1])],
            [16])[0]

        nuslab = _scatter_add2(avdx, dst, _N, 16)

        x16 = _rowcall(_xup_body, _N, _NBLK, [x16, nuslab, bgv, s2, xA16, xB16],
                       [], [16])[0]

    out = pl.pallas_call(
        _final_body,
        grid=(1,),
        in_specs=[pl.BlockSpec((_N, 16), lambda i: (0, 0))] * 3
        + [pl.BlockSpec((_N, 1), lambda i: (0, 0))],
        out_specs=pl.BlockSpec((_N, 16), lambda i: (0, 0)),
        out_shape=jax.ShapeDtypeStruct((_N, 16), jnp.float32),
    )(x16, xA16, xB16, s2)
    return out[:, :3]


# speedup vs baseline: 3.2069x; 3.0079x over previous
import functools

import jax
import jax.numpy as jnp
import numpy as np
from jax import lax
from jax.experimental import pallas as pl
from jax.experimental.pallas import tpu as pltpu
from jax.experimental.pallas import tpu_sc as plsc

_N = 10000
_E = 320000
_N_FREQ = 8
_N_LAYERS = 3
_D_CUTOFF = 5.0
_N_RBF = 10
_N_Z = 100
_STATE = 80

_NBLK = 2000
_EBLK = 3200

_GAMMA = float(1.0 / (_D_CUTOFF / (_N_RBF - 1)) ** 2)


def _rbf(d):
    step = _D_CUTOFF / (_N_RBF - 1)
    c = lax.broadcasted_iota(jnp.int32, (1, _N_RBF), 1).astype(jnp.float32) * step
    return jnp.exp(-_GAMMA * (d - c) ** 2)


def _gelu(x):
    return jax.nn.gelu(x)


def _rowcall(fn, nrows, block, row_ins, full_ins, out_cols):
    grid = (nrows // block,)

    def mk_row_spec(a):
        nd = a.ndim
        if nd == 3:
            return pl.BlockSpec((a.shape[0], block, a.shape[2]),
                                lambda i: (0, i, 0))
        return pl.BlockSpec((block,) + a.shape[1:],
                            lambda i, _nd=nd: (i,) + (0,) * (_nd - 1))

    def mk_full_spec(a):
        nd = a.ndim
        return pl.BlockSpec(a.shape, lambda i, _nd=nd: (0,) * _nd)

    in_specs = [mk_row_spec(a) for a in row_ins] + [mk_full_spec(a) for a in full_ins]
    out_shapes = []
    out_specs = []
    for c in out_cols:
        if isinstance(c, tuple):
            w = c[1]
            out_shapes.append(jax.ShapeDtypeStruct((nrows // _SC_B, _SC_B, w), jnp.float32))
            out_specs.append(pl.BlockSpec((block // _SC_B, _SC_B, w), lambda i: (i, 0, 0)))
        else:
            out_shapes.append(jax.ShapeDtypeStruct((nrows, c), jnp.float32))
            out_specs.append(pl.BlockSpec((block, c), lambda i: (i, 0)))
    return pl.pallas_call(
        fn,
        grid=grid,
        in_specs=in_specs,
        out_specs=out_specs,
        out_shape=out_shapes,
    )(*row_ins, *full_ins)


def _mm(a, w):
    return jnp.dot(a, w, preferred_element_type=jnp.float32)


def _node0_body(z_ref, s_ref, Wi0, bi0, Wi1, bi1, WA0, bA0, WB0, bB0,
                tab_ref, h0_ref, semb_ref):
    z = z_ref[...]
    onehot = (lax.broadcasted_iota(jnp.int32, (z.shape[0], _N_Z), 1) == z).astype(jnp.float32)
    ie = _gelu(_mm(onehot, Wi0[...]) + bi0[...])
    atom = _mm(ie, Wi1[...]) + bi1[...]
    tA = _gelu(_mm(onehot, WA0[...]) + bA0[...])
    tB = _gelu(_mm(onehot, WB0[...]) + bB0[...])
    sv = s_ref[...]
    freqs = lax.broadcasted_iota(jnp.int32, (1, _N_FREQ), 1).astype(jnp.float32) + 1.0
    ang = np.pi * sv * freqs
    semb = jnp.concatenate([jnp.sin(ang), jnp.cos(ang)], axis=1)
    tab_ref[...] = jnp.concatenate([tA, tB], axis=1)
    h0_ref[...] = jnp.concatenate([atom, semb], axis=1)
    semb_ref[...] = semb


def _split_body(t_ref, a_ref, b_ref):
    z = jnp.zeros((t_ref.shape[0], 64), jnp.float32)
    a_ref[...] = jnp.concatenate([t_ref[:, :64], z], axis=1).reshape(a_ref.shape)
    b_ref[...] = jnp.concatenate([t_ref[:, 64:], z], axis=1).reshape(b_ref.shape)


def _node1_body(tab_ref, aggA_ref, aggB_ref, semb_ref, h0_ref, s_ref, xA_ref, xB_ref,
                W1At, W1Ab, b1A, W1Bt, W1Bb, b1B,
                Wf1_0, bf1_0, Wf2_0, bf2_0,
                Wf1_1, bf1_1, Wf2_1, bf2_1,
                Wf1_2, bf1_2, Wf2_2, bf2_2,
                Wms0, Wmd0, bm0,
                f0_ref, f1_ref, f2_ref, x_ref, ts_ref, td_ref):
    aggA = aggA_ref[0, :, :64] + aggA_ref[1, :, :64]
    aggB = aggB_ref[0, :, :64] + aggB_ref[1, :, :64]
    tA = tab_ref[:, :64]
    tB = tab_ref[:, 64:]
    hA = _gelu(_mm(tA, W1At[...]) + _mm(aggA, W1Ab[...]) + b1A[...])
    hB = _gelu(_mm(tB, W1Bt[...]) + _mm(aggB, W1Bb[...]) + b1B[...])
    fi = jnp.concatenate([hA, hB, semb_ref[...]], axis=1)
    f0_ref[...] = _mm(_gelu(_mm(fi, Wf1_0[...]) + bf1_0[...]), Wf2_0[...]) + bf2_0[...]
    f1_ref[...] = _mm(_gelu(_mm(fi, Wf1_1[...]) + bf1_1[...]), Wf2_1[...]) + bf2_1[...]
    f2_ref[...] = _mm(_gelu(_mm(fi, Wf1_2[...]) + bf1_2[...]), Wf2_2[...]) + bf2_2[...]
    h0 = h0_ref[...]
    sv = s_ref[...]
    x16 = (1.0 - sv) * xA_ref[...] + sv * xB_ref[...]
    x_ref[...] = x16
    xa4 = xA_ref[:, :4]
    xb4 = xB_ref[:, :4]
    x4 = x16[:, :4]
    padc = jnp.zeros((x4.shape[0], 52), jnp.float32)
    ts_ref[...] = jnp.concatenate([_mm(h0, Wms0[...]), x4, xa4, xb4, padc], axis=1)
    td_ref[...] = jnp.concatenate([_mm(h0, Wmd0[...]) + bm0[...], x4, xa4, xb4, padc], axis=1)


def _edge_dyn(dx4, ef, We_s, We_d):
    dist = jnp.sqrt(jnp.sum(dx4 * dx4, axis=1, keepdims=True) + 1e-12)
    efd = jnp.concatenate([dist, dist * dist, _rbf(dist)], axis=1)
    return _mm(ef, We_s) + _mm(efd, We_d)


def _msg0_body(gs_ref, gd_ref, bond_ref,
               We_s, We_d, Wm1, bm1, Wm2, bm2,
               msg_ref, dx_ref, ef_ref):
    gs = gs_ref[...]
    gd = gd_ref[...]
    dA4 = gs[:, 68:72] - gd[:, 68:72]
    dB4 = gs[:, 72:76] - gd[:, 72:76]
    dA = jnp.sqrt(jnp.sum(dA4 * dA4, axis=1, keepdims=True) + 1e-12)
    dB = jnp.sqrt(jnp.sum(dB4 * dB4, axis=1, keepdims=True) + 1e-12)
    zp = jnp.zeros((gs.shape[0], 7), jnp.float32)
    ef = jnp.concatenate([bond_ref[...], dA, dB, dA - dB, _rbf(dA), _rbf(dB), zp], axis=1)
    ef_ref[...] = ef
    dx4 = gs[:, 64:68] - gd[:, 64:68]
    pre = gs[:, :64] + gd[:, :64] + _edge_dyn(dx4, ef, We_s[...], We_d[...])
    z = _gelu(pre)
    z = _gelu(_mm(z, Wm1[...]) + bm1[...])
    msgv = _mm(z, Wm2[...]) + bm2[...]
    msg_ref[...] = jnp.concatenate(
        [msgv, jnp.zeros((msgv.shape[0], 64), jnp.float32)], axis=1).reshape(msg_ref.shape)
    dx_ref[...] = jnp.concatenate([dx4, jnp.zeros((gs.shape[0], 12), jnp.float32)], axis=1)


def _msg_body(gs_ref, gd_ref, ef_ref,
              We_s, We_d, Wm1, bm1, Wm2, bm2,
              msg_ref, dx_ref):
    gs = gs_ref[...]
    gd = gd_ref[...]
    dx4 = gs[:, 64:68] - gd[:, 64:68]
    pre = gs[:, :64] + gd[:, :64] + _edge_dyn(dx4, ef_ref[...], We_s[...], We_d[...])
    z = _gelu(pre)
    z = _gelu(_mm(z, Wm1[...]) + bm1[...])
    msgv = _mm(z, Wm2[...]) + bm2[...]
    msg_ref[...] = jnp.concatenate(
        [msgv, jnp.zeros((msgv.shape[0], 64), jnp.float32)], axis=1).reshape(msg_ref.shape)
    dx_ref[...] = jnp.concatenate([dx4, jnp.zeros((gs.shape[0], 12), jnp.float32)], axis=1)


def _state_body(h_ref, ms_ref, f_ref,
                Ws_h, Ws_m, bs1, Ws2, bs2, Ws3, bs3,
                Wa_s, Wa_d, ba,
                Wb1, bb1, Wb2, bb2, Wb3, bb3,
                Wg1, bg1, Wg2, bg2, Wg3, bg3,
                Wms, Wmd, bm,
                h_out, ta_ref, bg_ref, ums_ref, umd_ref):
    h = h_ref[...]
    nm = ms_ref[0, :, :64] + ms_ref[1, :, :64]
    u = _gelu(_mm(h, Ws_h[...]) + _mm(nm, Ws_m[...]) + bs1[...])
    u = _gelu(_mm(u, Ws2[...]) + bs2[...])
    h2 = h + _mm(u, Ws3[...]) + bs3[...]
    g = f_ref[:, :_STATE]
    b = f_ref[:, _STATE:]
    h2 = _gelu((1.0 + g) * h2 + b)
    h_out[...] = h2
    ta_ref[...] = jnp.concatenate([_mm(h2, Wa_s[...]), _mm(h2, Wa_d[...]) + ba[...]], axis=1)
    zb = _gelu(_mm(h2, Wb1[...]) + bb1[...])
    zb = _gelu(_mm(zb, Wb2[...]) + bb2[...])
    bet = _mm(zb, Wb3[...]) + bb3[...]
    zg = _gelu(_mm(h2, Wg1[...]) + bg1[...])
    zg = _gelu(_mm(zg, Wg2[...]) + bg2[...])
    gam = _mm(zg, Wg3[...]) + bg3[...]
    bg_ref[...] = jnp.concatenate([bet, gam], axis=1)
    ums_ref[...] = _mm(h2, Wms[...])
    umd_ref[...] = _mm(h2, Wmd[...]) + bm[...]


def _alpha_body(gs_ref, gd_ref, dx_ref, ef_ref,
                Wae_s, Wae_d, Wa1, ba1, Wa2, ba2,
                av_ref):
    dx16 = dx_ref[...]
    pre = (gs_ref[:, :64] + gd_ref[:, 64:]
           + _edge_dyn(dx16[:, :4], ef_ref[...], Wae_s[...], Wae_d[...]))
    z = _gelu(pre)
    z = _gelu(_mm(z, Wa1[...]) + ba1[...])
    alpha = _mm(z, Wa2[...]) + ba2[...]
    av_ref[...] = jnp.concatenate(
        [alpha * dx16, jnp.zeros((dx16.shape[0], 112), jnp.float32)],
        axis=1).reshape(av_ref.shape)


def _xup_body(x_ref, nu_ref, bg_ref, s_ref, xA_ref, xB_ref, ums_ref, umd_ref,
              xo_ref, ts_ref, td_ref):
    x = x_ref[...]
    nu = nu_ref[0, :, :16] + nu_ref[1, :, :16]
    bet = bg_ref[:, 0:1]
    gam = bg_ref[:, 1:2]
    sv = s_ref[...]
    xA = xA_ref[...]
    xB = xB_ref[...]
    xn = x + nu + bet * (1.0 - sv) * (xA - x) + gam * sv * (xB - x)
    xo_ref[...] = xn
    x4 = xn[:, :4]
    xa4 = xA[:, :4]
    xb4 = xB[:, :4]
    padc = jnp.zeros((x4.shape[0], 52), jnp.float32)
    ts_ref[...] = jnp.concatenate([ums_ref[...], x4, xa4, xb4, padc], axis=1)
    td_ref[...] = jnp.concatenate([umd_ref[...], x4, xa4, xb4, padc], axis=1)


def _final_body(x_ref, xA_ref, xB_ref, s_ref, out_ref):
    sv = s_ref[...]
    base = (1.0 - sv) * xA_ref[...] + sv * xB_ref[...]
    corr = x_ref[...] - base
    xf = base + sv * (1.0 - sv) * corr
    out_ref[...] = xf - jnp.sum(xf, axis=0, keepdims=True) * (1.0 / _N)


_NC = 2
_NS = 16
_NW = _NC * _NS
_SC_C = 80
_SC_K = 8
_SC_H = 4
_SC_G = _SC_C * _SC_K
_SC_B = _SC_C * _SC_H
_NGRP_TOT = _E // _SC_G
_GRP_BASE = _NGRP_TOT // _NW
_GRP_EXTRA = _NGRP_TOT - _GRP_BASE * _NW
_RPT = 624
_ZR = 16


@functools.cache
def _sc_mesh():
    return plsc.VectorSubcoreMesh(core_axis_name="c", subcore_axis_name="s")


def _sc_wid():
    return lax.axis_index("s") * _NC + lax.axis_index("c")


def _sc_grp_range(wid):
    g0 = wid * _GRP_BASE + jnp.minimum(wid, _GRP_EXTRA)
    ng = _GRP_BASE + (wid < _GRP_EXTRA).astype(jnp.int32)
    return g0, ng


def _sc_gather_multi(tables, idx3ds):
    K = len(tables)
    widths = [int(t.shape[1]) for t in tables]
    out_type = [jax.ShapeDtypeStruct((_E, w), jnp.float32) for w in widths]
    scratch = ([pltpu.VMEM((_SC_K, _SC_C), jnp.int32) for _ in range(K)]
               + [pltpu.VMEM((_SC_B, w), jnp.float32) for w in widths]
               + [pltpu.SemaphoreType.DMA for _ in range(K)])

    @functools.partial(pl.kernel, out_type=out_type, mesh=_sc_mesh(),
                       scratch_types=scratch)
    def run(*refs):
        t_refs = refs[:K]
        i_refs = refs[K:2 * K]
        o_refs = refs[2 * K:3 * K]
        idxb = refs[3 * K:4 * K]
        rowb = refs[4 * K:5 * K]
        sems = refs[5 * K:6 * K]
        g0, ng = _sc_grp_range(_sc_wid())

        def group(g, _):
            grp = g0 + g
            for k in range(K):
                pltpu.sync_copy(i_refs[k].at[grp], idxb[k])
            for sub in range(_SC_K // _SC_H):
                for k in range(K):
                    descs = [pltpu.async_copy(
                        t_refs[k].at[idxb[k].at[sub * _SC_H + j]],
                        rowb[k].at[pl.ds(j * _SC_C, _SC_C)], sems[k])
                        for j in range(_SC_H)]
                    for d in descs:
                        d.wait()
                    pltpu.sync_copy(
                        rowb[k],
                        o_refs[k].at[pl.ds(grp * _SC_G + sub * _SC_B, _SC_B)])
            return ()

        lax.fori_loop(0, ng, group, (), unroll=False)

    return run(*tables, *idx3ds)


def _sc_zero_accum(z_ref, zbuf, accum, sid):
    pltpu.sync_copy(z_ref, zbuf)

    def zcp(j, _):
        pltpu.sync_copy(zbuf, accum.at[pl.ds(sid * _RPT + j * _ZR, _ZR)])
        return ()

    lax.fori_loop(0, _RPT // _ZR, zcp, (), unroll=False)

    @pl.when(sid == _NS - 1)
    def _():
        pltpu.sync_copy(zbuf, accum.at[pl.ds(_NS * _RPT, _ZR)])


def _sc_accum_out(accum, o_ref, cid, sid):
    @pl.when(sid < _NS - 1)
    def _():
        pltpu.sync_copy(accum.at[pl.ds(sid * _RPT, _RPT)],
                        o_ref.at[cid, pl.ds(sid * _RPT, _RPT)])

    @pl.when(sid == _NS - 1)
    def _():
        pltpu.sync_copy(accum.at[pl.ds((_NS - 1) * _RPT, _RPT + _ZR)],
                        o_ref.at[cid, pl.ds((_NS - 1) * _RPT, _RPT + _ZR)])


def _sc_scatter_add2(data, idx3d):
    w = 128
    out_type = jax.ShapeDtypeStruct((_NC, _N, w), jnp.float32)
    scratch = [pltpu.VMEM((_SC_K, _SC_C), jnp.int32),
               pltpu.VMEM((_SC_B, 128), jnp.float32),
               pltpu.VMEM((_ZR, w), jnp.float32),
               pltpu.VMEM_SHARED((_N, w), jnp.float32)]

    @functools.partial(pl.kernel, out_type=out_type, mesh=_sc_mesh(),
                       scratch_types=scratch)
    def run(d_ref, i_ref, z_ref, o_ref, idxb, rowb, zbuf, accum):
        cid = lax.axis_index("c")
        sid = lax.axis_index("s")
        g0, ng = _sc_grp_range(_sc_wid())
        _sc_zero_accum(z_ref, zbuf, accum, sid)
        plsc.subcore_barrier()

        def group(g, _):
            grp = g0 + g
            pltpu.sync_copy(i_ref.at[grp], idxb)
            for h in range(2):
                pltpu.sync_copy(d_ref.at[2 * grp + h], rowb)
                for j in range(_SC_H):
                    pltpu.sync_copy(rowb.at[pl.ds(j * _SC_C, _SC_C)],
                                    accum.at[idxb.at[h * _SC_H + j]], add=True)
            return ()

        lax.fori_loop(0, ng, group, (), unroll=False)
        plsc.subcore_barrier()
        _sc_accum_out(accum, o_ref, cid, sid)

    return run(data, idx3d, jnp.zeros((_ZR, w), jnp.float32))


def _pack_edge_first(W, b):
    Ws = W[:_STATE]
    Wd = W[_STATE:2 * _STATE]
    We = W[2 * _STATE:]
    stat = jnp.concatenate([We[0][None], We[1][None], We[4][None], We[5][None],
                            We[6][None], We[17:27], We[27:37],
                            jnp.zeros((7, We.shape[1]), jnp.float32)], axis=0)
    dyn = jnp.concatenate([We[2][None], We[3][None], We[7:17]], axis=0)
    return Ws, Wd, stat, dyn, b[None, :]


def _r2(b):
    return b[None, :]


def kernel(xA_pos, xB_pos, s, is_bond_A, is_bond_B, params, Z, edge_index):
    s2 = s.reshape(_N, 1)
    z2 = Z.reshape(_N, 1).astype(jnp.int32)
    src3d = edge_index[0].astype(jnp.int32).reshape(_E // _SC_G, _SC_K, _SC_C)
    dst3d = edge_index[1].astype(jnp.int32).reshape(_E // _SC_G, _SC_K, _SC_C)
    pad13 = jnp.zeros((_N, 13), jnp.float32)
    xA16 = jnp.concatenate([xA_pos, pad13], axis=1)
    xB16 = jnp.concatenate([xB_pos, pad13], axis=1)
    bond2 = jnp.stack([is_bond_A, is_bond_B], axis=1)

    p = params
    Wi0, bi0 = p['info'][0]
    Wi1, bi1 = p['info'][1]
    WA0, bA0 = p['embA'][0]
    WB0, bB0 = p['embB'][0]
    W1A, b1A = p['embA'][1]
    W1B, b1B = p['embB'][1]

    tab, h0, semb = _rowcall(
        _node0_body, _N, _NBLK, [z2, s2],
        [Wi0, _r2(bi0), Wi1, _r2(bi1), WA0, _r2(bA0), WB0, _r2(bB0)],
        [128, 80, 16])

    tsrc = _sc_gather_multi([tab], [src3d])[0]
    eA, eB = _rowcall(_split_body, _E, _EBLK, [tsrc], [], [('e3', 128), ('e3', 128)])
    aggA = _sc_scatter_add2(eA, dst3d)
    aggB = _sc_scatter_add2(eB, dst3d)

    lw = p['layers']
    film_w = []
    for l in range(_N_LAYERS):
        Wf1, bf1 = lw[l]['film'][0]
        Wf2, bf2 = lw[l]['film'][1]
        film_w += [Wf1, _r2(bf1), Wf2, _r2(bf2)]
    msg_first = [_pack_edge_first(*lw[l]['msg'][0]) for l in range(_N_LAYERS)]
    alpha_first = [_pack_edge_first(*lw[l]['alpha'][0]) for l in range(_N_LAYERS)]

    f0, f1, f2, x16, ts, td = _rowcall(
        _node1_body, _N, _NBLK, [tab, aggA, aggB, semb, h0, s2, xA16, xB16],
        [W1A[:64], W1A[64:], _r2(b1A), W1B[:64], W1B[64:], _r2(b1B)]
        + film_w
        + [msg_first[0][0], msg_first[0][1], msg_first[0][4]],
        [160, 160, 160, 16, 128, 128])
    films = [f0, f1, f2]

    h = h0
    ef = None
    dx16 = None
    for l in range(_N_LAYERS):
        lp = lw[l]
        _, _, We_s, We_d, _ = msg_first[l]
        Wa_s, Wa_d, Wae_s, Wae_d, ba = alpha_first[l]

        gs, gd = _sc_gather_multi([ts, td], [src3d, dst3d])
        if l == 0:
            msg, dx16, ef = _rowcall(
                _msg0_body, _E, _EBLK, [gs, gd, bond2],
                [We_s, We_d, lp['msg'][1][0], _r2(lp['msg'][1][1]),
                 lp['msg'][2][0], _r2(lp['msg'][2][1])],
                [('e3', 128), 16, 32])
        else:
            msg, dx16 = _rowcall(
                _msg_body, _E, _EBLK, [gs, gd, ef],
                [We_s, We_d, lp['msg'][1][0], _r2(lp['msg'][1][1]),
                 lp['msg'][2][0], _r2(lp['msg'][2][1])],
                [('e3', 128), 16])

        mslab = _sc_scatter_add2(msg, dst3d)

        Ws0, bs0 = lp['state'][0]
        next_l = min(l + 1, _N_LAYERS - 1)
        Wms_n, Wmd_n, _, _, bm_n = msg_first[next_l]
        h, ta, bgv, ums, umd = _rowcall(
            _state_body, _N, _NBLK, [h, mslab, films[l]],
            [Ws0[:_STATE], Ws0[_STATE:], _r2(bs0),
             lp['state'][1][0], _r2(lp['state'][1][1]),
             lp['state'][2][0], _r2(lp['state'][2][1]),
             Wa_s, Wa_d, ba,
             lp['beta'][0][0], _r2(lp['beta'][0][1]),
             lp['beta'][1][0], _r2(lp['beta'][1][1]),
             lp['beta'][2][0], _r2(lp['beta'][2][1]),
             lp['gamma'][0][0], _r2(lp['gamma'][0][1]),
             lp['gamma'][1][0], _r2(lp['gamma'][1][1]),
             lp['gamma'][2][0], _r2(lp['gamma'][2][1]),
             Wms_n, Wmd_n, bm_n],
            [80, 128, 2, 64, 64])

        gas, gad = _sc_gather_multi([ta, ta], [src3d, dst3d])
        avdx = _rowcall(
            _alpha_body, _E, _EBLK, [gas, gad, dx16, ef],
            [Wae_s, Wae_d, lp['alpha'][1][0], _r2(lp['alpha'][1][1]),
             lp['alpha'][2][0], _r2(lp['alpha'][2][1])],
            [('e3', 128)])[0]

        nuslab = _sc_scatter_add2(avdx, dst3d)

        x16, ts, td = _rowcall(
            _xup_body, _N, _NBLK, [x16, nuslab, bgv, s2, xA16, xB16, ums, umd],
            [], [16, 128, 128])

    out = pl.pallas_call(
        _final_body,
        grid=(1,),
        in_specs=[pl.BlockSpec((_N, 16), lambda i: (0, 0))] * 3
        + [pl.BlockSpec((_N, 1), lambda i: (0, 0))],
        out_specs=pl.BlockSpec((_N, 16), lambda i: (0, 0)),
        out_shape=jax.ShapeDtypeStruct((_N, 16), jnp.float32),
    )(x16, xA16, xB16, s2)
    return out[:, :3]


# speedup vs baseline: 3.2883x; 1.0254x over previous
import functools

import jax
import jax.numpy as jnp
import numpy as np
from jax import lax
from jax.experimental import pallas as pl
from jax.experimental.pallas import tpu as pltpu
from jax.experimental.pallas import tpu_sc as plsc

_N = 10000
_E = 320000
_N_FREQ = 8
_N_LAYERS = 3
_D_CUTOFF = 5.0
_N_RBF = 10
_N_Z = 100
_STATE = 80

_NBLK = 2000
_EBLK = 3200

_GAMMA = float(1.0 / (_D_CUTOFF / (_N_RBF - 1)) ** 2)


def _rbf(d):
    step = _D_CUTOFF / (_N_RBF - 1)
    c = lax.broadcasted_iota(jnp.int32, (1, _N_RBF), 1).astype(jnp.float32) * step
    return jnp.exp(-_GAMMA * (d - c) ** 2)


def _gelu(x):
    return jax.nn.gelu(x)


def _rowcall(fn, nrows, block, row_ins, full_ins, out_cols):
    grid = (nrows // block,)

    def mk_row_spec(a):
        nd = a.ndim
        if nd == 3:
            return pl.BlockSpec((a.shape[0], block, a.shape[2]),
                                lambda i: (0, i, 0))
        return pl.BlockSpec((block,) + a.shape[1:],
                            lambda i, _nd=nd: (i,) + (0,) * (_nd - 1))

    def mk_full_spec(a):
        nd = a.ndim
        return pl.BlockSpec(a.shape, lambda i, _nd=nd: (0,) * _nd)

    in_specs = [mk_row_spec(a) for a in row_ins] + [mk_full_spec(a) for a in full_ins]
    out_shapes = []
    out_specs = []
    for c in out_cols:
        if isinstance(c, tuple):
            w = c[1]
            out_shapes.append(jax.ShapeDtypeStruct((nrows // _SC_B, _SC_B, w), jnp.float32))
            out_specs.append(pl.BlockSpec((block // _SC_B, _SC_B, w), lambda i: (i, 0, 0)))
        else:
            out_shapes.append(jax.ShapeDtypeStruct((nrows, c), jnp.float32))
            out_specs.append(pl.BlockSpec((block, c), lambda i: (i, 0)))
    return pl.pallas_call(
        fn,
        grid=grid,
        in_specs=in_specs,
        out_specs=out_specs,
        out_shape=out_shapes,
    )(*row_ins, *full_ins)


def _mm(a, w):
    return jnp.dot(a, w, preferred_element_type=jnp.float32)


def _node0_body(z_ref, s_ref, Wi0, bi0, Wi1, bi1, WA0, bA0, WB0, bB0,
                tab_ref, h0_ref, semb_ref):
    z = z_ref[...]
    onehot = (lax.broadcasted_iota(jnp.int32, (z.shape[0], _N_Z), 1) == z).astype(jnp.float32)
    ie = _gelu(_mm(onehot, Wi0[...]) + bi0[...])
    atom = _mm(ie, Wi1[...]) + bi1[...]
    tA = _gelu(_mm(onehot, WA0[...]) + bA0[...])
    tB = _gelu(_mm(onehot, WB0[...]) + bB0[...])
    sv = s_ref[...]
    freqs = lax.broadcasted_iota(jnp.int32, (1, _N_FREQ), 1).astype(jnp.float32) + 1.0
    ang = np.pi * sv * freqs
    semb = jnp.concatenate([jnp.sin(ang), jnp.cos(ang)], axis=1)
    tab_ref[...] = jnp.concatenate([tA, tB], axis=1)
    h0_ref[...] = jnp.concatenate([atom, semb], axis=1)
    semb_ref[...] = semb


def _split_body(t_ref, a_ref, b_ref):
    z = jnp.zeros((t_ref.shape[0], 64), jnp.float32)
    a_ref[...] = jnp.concatenate([t_ref[:, :64], z], axis=1).reshape(a_ref.shape)
    b_ref[...] = jnp.concatenate([t_ref[:, 64:], z], axis=1).reshape(b_ref.shape)


def _node1_body(tab_ref, aggA_ref, aggB_ref, semb_ref, h0_ref, s_ref, xA_ref, xB_ref,
                W1At, W1Ab, b1A, W1Bt, W1Bb, b1B,
                Wf1_0, bf1_0, Wf2_0, bf2_0,
                Wf1_1, bf1_1, Wf2_1, bf2_1,
                Wf1_2, bf1_2, Wf2_2, bf2_2,
                Wms0, Wmd0, bm0,
                f0_ref, f1_ref, f2_ref, x_ref, ts_ref, td_ref):
    aggA = aggA_ref[0, :, :64] + aggA_ref[1, :, :64]
    aggB = aggB_ref[0, :, :64] + aggB_ref[1, :, :64]
    tA = tab_ref[:, :64]
    tB = tab_ref[:, 64:]
    hA = _gelu(_mm(tA, W1At[...]) + _mm(aggA, W1Ab[...]) + b1A[...])
    hB = _gelu(_mm(tB, W1Bt[...]) + _mm(aggB, W1Bb[...]) + b1B[...])
    fi = jnp.concatenate([hA, hB, semb_ref[...]], axis=1)
    f0_ref[...] = _mm(_gelu(_mm(fi, Wf1_0[...]) + bf1_0[...]), Wf2_0[...]) + bf2_0[...]
    f1_ref[...] = _mm(_gelu(_mm(fi, Wf1_1[...]) + bf1_1[...]), Wf2_1[...]) + bf2_1[...]
    f2_ref[...] = _mm(_gelu(_mm(fi, Wf1_2[...]) + bf1_2[...]), Wf2_2[...]) + bf2_2[...]
    h0 = h0_ref[...]
    sv = s_ref[...]
    x16 = (1.0 - sv) * xA_ref[...] + sv * xB_ref[...]
    x_ref[...] = x16
    xa4 = xA_ref[:, :4]
    xb4 = xB_ref[:, :4]
    x4 = x16[:, :4]
    padc = jnp.zeros((x4.shape[0], 52), jnp.float32)
    ts_ref[...] = jnp.concatenate([_mm(h0, Wms0[...]), x4, xa4, xb4, padc], axis=1)
    td_ref[...] = jnp.concatenate([_mm(h0, Wmd0[...]) + bm0[...], x4, xa4, xb4, padc], axis=1)


def _edge_dyn(dx4, ef, We_s, We_d):
    dist = jnp.sqrt(jnp.sum(dx4 * dx4, axis=1, keepdims=True) + 1e-12)
    efd = jnp.concatenate([dist, dist * dist, _rbf(dist)], axis=1)
    return _mm(ef, We_s) + _mm(efd, We_d)


def _msg0_body(gs_ref, gd_ref, bond_ref,
               We_s, We_d, Wm1, bm1, Wm2, bm2,
               msg_ref, dx_ref, ef_ref):
    gs = gs_ref[...]
    gd = gd_ref[...]
    dA4 = gs[:, 68:72] - gd[:, 68:72]
    dB4 = gs[:, 72:76] - gd[:, 72:76]
    dA = jnp.sqrt(jnp.sum(dA4 * dA4, axis=1, keepdims=True) + 1e-12)
    dB = jnp.sqrt(jnp.sum(dB4 * dB4, axis=1, keepdims=True) + 1e-12)
    zp = jnp.zeros((gs.shape[0], 7), jnp.float32)
    ef = jnp.concatenate([bond_ref[...], dA, dB, dA - dB, _rbf(dA), _rbf(dB), zp], axis=1)
    ef_ref[...] = ef
    dx4 = gs[:, 64:68] - gd[:, 64:68]
    pre = gs[:, :64] + gd[:, :64] + _edge_dyn(dx4, ef, We_s[...], We_d[...])
    z = _gelu(pre)
    z = _gelu(_mm(z, Wm1[...]) + bm1[...])
    msgv = _mm(z, Wm2[...]) + bm2[...]
    msg_ref[...] = jnp.concatenate(
        [msgv, jnp.zeros((msgv.shape[0], 64), jnp.float32)], axis=1).reshape(msg_ref.shape)
    dx_ref[...] = jnp.concatenate([dx4, jnp.zeros((gs.shape[0], 12), jnp.float32)], axis=1)


def _msg_body(gs_ref, gd_ref, ef_ref,
              We_s, We_d, Wm1, bm1, Wm2, bm2,
              msg_ref, dx_ref):
    gs = gs_ref[...]
    gd = gd_ref[...]
    dx4 = gs[:, 64:68] - gd[:, 64:68]
    pre = gs[:, :64] + gd[:, :64] + _edge_dyn(dx4, ef_ref[...], We_s[...], We_d[...])
    z = _gelu(pre)
    z = _gelu(_mm(z, Wm1[...]) + bm1[...])
    msgv = _mm(z, Wm2[...]) + bm2[...]
    msg_ref[...] = jnp.concatenate(
        [msgv, jnp.zeros((msgv.shape[0], 64), jnp.float32)], axis=1).reshape(msg_ref.shape)
    dx_ref[...] = jnp.concatenate([dx4, jnp.zeros((gs.shape[0], 12), jnp.float32)], axis=1)


def _state_body(h_ref, ms_ref, f_ref,
                Ws_h, Ws_m, bs1, Ws2, bs2, Ws3, bs3,
                Wa_s, Wa_d, ba,
                Wb1, bb1, Wb2, bb2, Wb3, bb3,
                Wg1, bg1, Wg2, bg2, Wg3, bg3,
                Wms, Wmd, bm,
                h_out, ta_ref, bg_ref, ums_ref, umd_ref):
    h = h_ref[...]
    nm = ms_ref[0, :, :64] + ms_ref[1, :, :64]
    u = _gelu(_mm(h, Ws_h[...]) + _mm(nm, Ws_m[...]) + bs1[...])
    u = _gelu(_mm(u, Ws2[...]) + bs2[...])
    h2 = h + _mm(u, Ws3[...]) + bs3[...]
    g = f_ref[:, :_STATE]
    b = f_ref[:, _STATE:]
    h2 = _gelu((1.0 + g) * h2 + b)
    h_out[...] = h2
    ta_ref[...] = jnp.concatenate([_mm(h2, Wa_s[...]), _mm(h2, Wa_d[...]) + ba[...]], axis=1)
    zb = _gelu(_mm(h2, Wb1[...]) + bb1[...])
    zb = _gelu(_mm(zb, Wb2[...]) + bb2[...])
    bet = _mm(zb, Wb3[...]) + bb3[...]
    zg = _gelu(_mm(h2, Wg1[...]) + bg1[...])
    zg = _gelu(_mm(zg, Wg2[...]) + bg2[...])
    gam = _mm(zg, Wg3[...]) + bg3[...]
    bg_ref[...] = jnp.concatenate([bet, gam], axis=1)
    ums_ref[...] = _mm(h2, Wms[...])
    umd_ref[...] = _mm(h2, Wmd[...]) + bm[...]


def _alpha_body(gs_ref, gd_ref, dx_ref, ef_ref,
                Wae_s, Wae_d, Wa1, ba1, Wa2, ba2,
                av_ref):
    dx16 = dx_ref[...]
    pre = (gs_ref[:, :64] + gd_ref[:, 64:]
           + _edge_dyn(dx16[:, :4], ef_ref[...], Wae_s[...], Wae_d[...]))
    z = _gelu(pre)
    z = _gelu(_mm(z, Wa1[...]) + ba1[...])
    alpha = _mm(z, Wa2[...]) + ba2[...]
    av_ref[...] = jnp.concatenate(
        [alpha * dx16, jnp.zeros((dx16.shape[0], 112), jnp.float32)],
        axis=1).reshape(av_ref.shape)


def _xup_body(x_ref, nu_ref, bg_ref, s_ref, xA_ref, xB_ref, ums_ref, umd_ref,
              xo_ref, ts_ref, td_ref):
    x = x_ref[...]
    nu = nu_ref[0, :, :16] + nu_ref[1, :, :16]
    bet = bg_ref[:, 0:1]
    gam = bg_ref[:, 1:2]
    sv = s_ref[...]
    xA = xA_ref[...]
    xB = xB_ref[...]
    xn = x + nu + bet * (1.0 - sv) * (xA - x) + gam * sv * (xB - x)
    xo_ref[...] = xn
    x4 = xn[:, :4]
    xa4 = xA[:, :4]
    xb4 = xB[:, :4]
    padc = jnp.zeros((x4.shape[0], 52), jnp.float32)
    ts_ref[...] = jnp.concatenate([ums_ref[...], x4, xa4, xb4, padc], axis=1)
    td_ref[...] = jnp.concatenate([umd_ref[...], x4, xa4, xb4, padc], axis=1)


def _final_body(x_ref, xA_ref, xB_ref, s_ref, out_ref):
    sv = s_ref[...]
    base = (1.0 - sv) * xA_ref[...] + sv * xB_ref[...]
    corr = x_ref[...] - base
    xf = base + sv * (1.0 - sv) * corr
    out_ref[...] = xf - jnp.sum(xf, axis=0, keepdims=True) * (1.0 / _N)


_NC = 2
_NS = 16
_NW = _NC * _NS
_SC_C = 80
_SC_K = 8
_SC_H = 4
_SC_G = _SC_C * _SC_K
_SC_B = _SC_C * _SC_H
_NGRP_TOT = _E // _SC_G
_GRP_BASE = _NGRP_TOT // _NW
_GRP_EXTRA = _NGRP_TOT - _GRP_BASE * _NW
_RPT = 624
_ZR = 16


@functools.cache
def _sc_mesh():
    return plsc.VectorSubcoreMesh(core_axis_name="c", subcore_axis_name="s")


def _sc_wid():
    return lax.axis_index("s") * _NC + lax.axis_index("c")


def _sc_grp_range(wid):
    g0 = wid * _GRP_BASE + jnp.minimum(wid, _GRP_EXTRA)
    ng = _GRP_BASE + (wid < _GRP_EXTRA).astype(jnp.int32)
    return g0, ng


def _sc_gather_multi(tables, idx3ds):
    K = len(tables)
    widths = [int(t.shape[1]) for t in tables]
    Q = 4
    R = _SC_C * 2
    out_type = [jax.ShapeDtypeStruct((_E, w), jnp.float32) for w in widths]
    scratch = ([pltpu.VMEM((_SC_K, _SC_C), jnp.int32) for _ in range(K)]
               + [pltpu.VMEM((2 * R, w), jnp.float32) for w in widths]
               + [pltpu.SemaphoreType.DMA for _ in range(K)]
               + [pltpu.SemaphoreType.DMA for _ in range(2 * K)])

    @functools.partial(pl.kernel, out_type=out_type, mesh=_sc_mesh(),
                       scratch_types=scratch)
    def run(*refs):
        t_refs = refs[:K]
        i_refs = refs[K:2 * K]
        o_refs = refs[2 * K:3 * K]
        idxb = refs[3 * K:4 * K]
        rowb = refs[4 * K:5 * K]
        gsems = refs[5 * K:6 * K]
        osems = refs[6 * K:8 * K]
        g0, ng = _sc_grp_range(_sc_wid())

        def group(g, _):
            grp = g0 + g
            for k in range(K):
                pltpu.sync_copy(i_refs[k].at[grp], idxb[k])
            for q in range(Q):
                p = q % 2
                for k in range(K):
                    buf = rowb[k].at[pl.ds(p * R, R)]
                    osem = osems[2 * k + p]
                    @pl.when((g > 0) | (q >= 2))
                    def _():
                        pltpu.make_async_copy(
                            buf, o_refs[k].at[pl.ds(0, R)], osem).wait()
                    descs = [pltpu.async_copy(
                        t_refs[k].at[idxb[k].at[2 * q + j]],
                        buf.at[pl.ds(j * _SC_C, _SC_C)], gsems[k])
                        for j in range(2)]
                    for d in descs:
                        d.wait()
                    pltpu.async_copy(
                        buf,
                        o_refs[k].at[pl.ds(grp * _SC_G + q * R, R)], osem)
            return ()

        lax.fori_loop(0, ng, group, (), unroll=False)
        for k in range(K):
            for p in range(2):
                pltpu.make_async_copy(
                    rowb[k].at[pl.ds(p * R, R)],
                    o_refs[k].at[pl.ds(0, R)], osems[2 * k + p]).wait()

    return run(*tables, *idx3ds)


def _sc_zero_accum(z_ref, zbuf, accum, sid):
    pltpu.sync_copy(z_ref, zbuf)

    def zcp(j, _):
        pltpu.sync_copy(zbuf, accum.at[pl.ds(sid * _RPT + j * _ZR, _ZR)])
        return ()

    lax.fori_loop(0, _RPT // _ZR, zcp, (), unroll=False)

    @pl.when(sid == _NS - 1)
    def _():
        pltpu.sync_copy(zbuf, accum.at[pl.ds(_NS * _RPT, _ZR)])


def _sc_accum_out(accum, o_ref, cid, sid):
    @pl.when(sid < _NS - 1)
    def _():
        pltpu.sync_copy(accum.at[pl.ds(sid * _RPT, _RPT)],
                        o_ref.at[cid, pl.ds(sid * _RPT, _RPT)])

    @pl.when(sid == _NS - 1)
    def _():
        pltpu.sync_copy(accum.at[pl.ds((_NS - 1) * _RPT, _RPT + _ZR)],
                        o_ref.at[cid, pl.ds((_NS - 1) * _RPT, _RPT + _ZR)])


def _sc_scatter_add2(data, idx3d):
    w = 128
    out_type = jax.ShapeDtypeStruct((_NC, _N, w), jnp.float32)
    scratch = [pltpu.VMEM((_SC_K, _SC_C), jnp.int32),
               pltpu.VMEM((_SC_B, 128), jnp.float32),
               pltpu.VMEM((_ZR, w), jnp.float32),
               pltpu.VMEM_SHARED((_N, w), jnp.float32),
               pltpu.SemaphoreType.DMA]

    @functools.partial(pl.kernel, out_type=out_type, mesh=_sc_mesh(),
                       scratch_types=scratch)
    def run(d_ref, i_ref, z_ref, o_ref, idxb, rowb, zbuf, accum, lsem):
        cid = lax.axis_index("c")
        sid = lax.axis_index("s")
        g0, ng = _sc_grp_range(_sc_wid())
        _sc_zero_accum(z_ref, zbuf, accum, sid)
        plsc.subcore_barrier()
        H = _SC_B // 2

        def group(g, _):
            grp = g0 + g
            pltpu.sync_copy(i_ref.at[grp], idxb)
            pltpu.async_copy(d_ref.at[2 * grp].at[pl.ds(0, H)],
                             rowb.at[pl.ds(0, H)], lsem)
            for q in range(4):
                p = q % 2
                buf = rowb.at[pl.ds(p * H, H)]
                pltpu.make_async_copy(d_ref.at[2 * grp].at[pl.ds(0, H)],
                                      buf, lsem).wait()
                if q < 3:
                    qs = q + 1
                    pltpu.async_copy(
                        d_ref.at[2 * grp + qs // 2].at[pl.ds((qs % 2) * H, H)],
                        rowb.at[pl.ds((1 - p) * H, H)], lsem)
                for j in range(2):
                    pltpu.sync_copy(buf.at[pl.ds(j * _SC_C, _SC_C)],
                                    accum.at[idxb.at[2 * q + j]], add=True)
            return ()

        lax.fori_loop(0, ng, group, (), unroll=False)
        plsc.subcore_barrier()
        _sc_accum_out(accum, o_ref, cid, sid)

    return run(data, idx3d, jnp.zeros((_ZR, w), jnp.float32))


def _pack_edge_first(W, b):
    Ws = W[:_STATE]
    Wd = W[_STATE:2 * _STATE]
    We = W[2 * _STATE:]
    stat = jnp.concatenate([We[0][None], We[1][None], We[4][None], We[5][None],
                            We[6][None], We[17:27], We[27:37],
                            jnp.zeros((7, We.shape[1]), jnp.float32)], axis=0)
    dyn = jnp.concatenate([We[2][None], We[3][None], We[7:17]], axis=0)
    return Ws, Wd, stat, dyn, b[None, :]


def _r2(b):
    return b[None, :]


def kernel(xA_pos, xB_pos, s, is_bond_A, is_bond_B, params, Z, edge_index):
    s2 = s.reshape(_N, 1)
    z2 = Z.reshape(_N, 1).astype(jnp.int32)
    src3d = edge_index[0].astype(jnp.int32).reshape(_E // _SC_G, _SC_K, _SC_C)
    dst3d = edge_index[1].astype(jnp.int32).reshape(_E // _SC_G, _SC_K, _SC_C)
    pad13 = jnp.zeros((_N, 13), jnp.float32)
    xA16 = jnp.concatenate([xA_pos, pad13], axis=1)
    xB16 = jnp.concatenate([xB_pos, pad13], axis=1)
    bond2 = jnp.stack([is_bond_A, is_bond_B], axis=1)

    p = params
    Wi0, bi0 = p['info'][0]
    Wi1, bi1 = p['info'][1]
    WA0, bA0 = p['embA'][0]
    WB0, bB0 = p['embB'][0]
    W1A, b1A = p['embA'][1]
    W1B, b1B = p['embB'][1]

    tab, h0, semb = _rowcall(
        _node0_body, _N, _NBLK, [z2, s2],
        [Wi0, _r2(bi0), Wi1, _r2(bi1), WA0, _r2(bA0), WB0, _r2(bB0)],
        [128, 80, 16])

    tsrc = _sc_gather_multi([tab], [src3d])[0]
    eA, eB = _rowcall(_split_body, _E, _EBLK, [tsrc], [], [('e3', 128), ('e3', 128)])
    aggA = _sc_scatter_add2(eA, dst3d)
    aggB = _sc_scatter_add2(eB, dst3d)

    lw = p['layers']
    film_w = []
    for l in range(_N_LAYERS):
        Wf1, bf1 = lw[l]['film'][0]
        Wf2, bf2 = lw[l]['film'][1]
        film_w += [Wf1, _r2(bf1), Wf2, _r2(bf2)]
    msg_first = [_pack_edge_first(*lw[l]['msg'][0]) for l in range(_N_LAYERS)]
    alpha_first = [_pack_edge_first(*lw[l]['alpha'][0]) for l in range(_N_LAYERS)]

    f0, f1, f2, x16, ts, td = _rowcall(
        _node1_body, _N, _NBLK, [tab, aggA, aggB, semb, h0, s2, xA16, xB16],
        [W1A[:64], W1A[64:], _r2(b1A), W1B[:64], W1B[64:], _r2(b1B)]
        + film_w
        + [msg_first[0][0], msg_first[0][1], msg_first[0][4]],
        [160, 160, 160, 16, 128, 128])
    films = [f0, f1, f2]

    h = h0
    ef = None
    dx16 = None
    for l in range(_N_LAYERS):
        lp = lw[l]
        _, _, We_s, We_d, _ = msg_first[l]
        Wa_s, Wa_d, Wae_s, Wae_d, ba = alpha_first[l]

        gs, gd = _sc_gather_multi([ts, td], [src3d, dst3d])
        if l == 0:
            msg, dx16, ef = _rowcall(
                _msg0_body, _E, _EBLK, [gs, gd, bond2],
                [We_s, We_d, lp['msg'][1][0], _r2(lp['msg'][1][1]),
                 lp['msg'][2][0], _r2(lp['msg'][2][1])],
                [('e3', 128), 16, 32])
        else:
            msg, dx16 = _rowcall(
                _msg_body, _E, _EBLK, [gs, gd, ef],
                [We_s, We_d, lp['msg'][1][0], _r2(lp['msg'][1][1]),
                 lp['msg'][2][0], _r2(lp['msg'][2][1])],
                [('e3', 128), 16])

        mslab = _sc_scatter_add2(msg, dst3d)

        Ws0, bs0 = lp['state'][0]
        next_l = min(l + 1, _N_LAYERS - 1)
        Wms_n, Wmd_n, _, _, bm_n = msg_first[next_l]
        h, ta, bgv, ums, umd = _rowcall(
            _state_body, _N, _NBLK, [h, mslab, films[l]],
            [Ws0[:_STATE], Ws0[_STATE:], _r2(bs0),
             lp['state'][1][0], _r2(lp['state'][1][1]),
             lp['state'][2][0], _r2(lp['state'][2][1]),
             Wa_s, Wa_d, ba,
             lp['beta'][0][0], _r2(lp['beta'][0][1]),
             lp['beta'][1][0], _r2(lp['beta'][1][1]),
             lp['beta'][2][0], _r2(lp['beta'][2][1]),
             lp['gamma'][0][0], _r2(lp['gamma'][0][1]),
             lp['gamma'][1][0], _r2(lp['gamma'][1][1]),
             lp['gamma'][2][0], _r2(lp['gamma'][2][1]),
             Wms_n, Wmd_n, bm_n],
            [80, 128, 2, 64, 64])

        gas, gad = _sc_gather_multi([ta, ta], [src3d, dst3d])
        avdx = _rowcall(
            _alpha_body, _E, _EBLK, [gas, gad, dx16, ef],
            [Wae_s, Wae_d, lp['alpha'][1][0], _r2(lp['alpha'][1][1]),
             lp['alpha'][2][0], _r2(lp['alpha'][2][1])],
            [('e3', 128)])[0]

        nuslab = _sc_scatter_add2(avdx, dst3d)

        x16, ts, td = _rowcall(
            _xup_body, _N, _NBLK, [x16, nuslab, bgv, s2, xA16, xB16, ums, umd],
            [], [16, 128, 128])

    out = pl.pallas_call(
        _final_body,
        grid=(1,),
        in_specs=[pl.BlockSpec((_N, 16), lambda i: (0, 0))] * 3
        + [pl.BlockSpec((_N, 1), lambda i: (0, 0))],
        out_specs=pl.BlockSpec((_N, 16), lambda i: (0, 0)),
        out_shape=jax.ShapeDtypeStruct((_N, 16), jnp.float32),
    )(x16, xA16, xB16, s2)
    return out[:, :3]


# speedup vs baseline: 3.4125x; 1.0378x over previous
import functools

import jax
import jax.numpy as jnp
import numpy as np
from jax import lax
from jax.experimental import pallas as pl
from jax.experimental.pallas import tpu as pltpu
from jax.experimental.pallas import tpu_sc as plsc

_N = 10000
_E = 320000
_N_FREQ = 8
_N_LAYERS = 3
_D_CUTOFF = 5.0
_N_RBF = 10
_N_Z = 100
_STATE = 80

_NBLK = 2000
_EBLK = 3200

_GAMMA = float(1.0 / (_D_CUTOFF / (_N_RBF - 1)) ** 2)


def _rbf(d):
    step = _D_CUTOFF / (_N_RBF - 1)
    c = lax.broadcasted_iota(jnp.int32, (1, _N_RBF), 1).astype(jnp.float32) * step
    return jnp.exp(-_GAMMA * (d - c) ** 2)


def _gelu(x):
    return jax.nn.gelu(x)


def _rowcall(fn, nrows, block, row_ins, full_ins, out_cols):
    grid = (nrows // block,)

    def mk_row_spec(a):
        nd = a.ndim
        if nd == 3:
            return pl.BlockSpec((a.shape[0], block, a.shape[2]),
                                lambda i: (0, i, 0))
        return pl.BlockSpec((block,) + a.shape[1:],
                            lambda i, _nd=nd: (i,) + (0,) * (_nd - 1))

    def mk_full_spec(a):
        nd = a.ndim
        return pl.BlockSpec(a.shape, lambda i, _nd=nd: (0,) * _nd)

    in_specs = [mk_row_spec(a) for a in row_ins] + [mk_full_spec(a) for a in full_ins]
    out_shapes = []
    out_specs = []
    for c in out_cols:
        if isinstance(c, tuple):
            w = c[1]
            out_shapes.append(jax.ShapeDtypeStruct((nrows // _SC_B, _SC_B, w), jnp.float32))
            out_specs.append(pl.BlockSpec((block // _SC_B, _SC_B, w), lambda i: (i, 0, 0)))
        else:
            out_shapes.append(jax.ShapeDtypeStruct((nrows, c), jnp.float32))
            out_specs.append(pl.BlockSpec((block, c), lambda i: (i, 0)))
    return pl.pallas_call(
        fn,
        grid=grid,
        in_specs=in_specs,
        out_specs=out_specs,
        out_shape=out_shapes,
    )(*row_ins, *full_ins)


def _mm(a, w):
    return jnp.dot(a, w, preferred_element_type=jnp.float32)


def _node0_body(z_ref, s_ref, Wi0, bi0, Wi1, bi1, WA0, bA0, WB0, bB0,
                tab_ref, h0_ref, semb_ref):
    z = z_ref[...]
    onehot = (lax.broadcasted_iota(jnp.int32, (z.shape[0], _N_Z), 1) == z).astype(jnp.float32)
    ie = _gelu(_mm(onehot, Wi0[...]) + bi0[...])
    atom = _mm(ie, Wi1[...]) + bi1[...]
    tA = _gelu(_mm(onehot, WA0[...]) + bA0[...])
    tB = _gelu(_mm(onehot, WB0[...]) + bB0[...])
    sv = s_ref[...]
    freqs = lax.broadcasted_iota(jnp.int32, (1, _N_FREQ), 1).astype(jnp.float32) + 1.0
    ang = np.pi * sv * freqs
    semb = jnp.concatenate([jnp.sin(ang), jnp.cos(ang)], axis=1)
    tab_ref[...] = jnp.concatenate([tA, tB], axis=1)
    h0_ref[...] = jnp.concatenate([atom, semb], axis=1)
    semb_ref[...] = semb


def _split_body(t_ref, a_ref, b_ref):
    z = jnp.zeros((t_ref.shape[0], 64), jnp.float32)
    a_ref[...] = jnp.concatenate([t_ref[:, :64], z], axis=1).reshape(a_ref.shape)
    b_ref[...] = jnp.concatenate([t_ref[:, 64:], z], axis=1).reshape(b_ref.shape)


def _node1_body(tab_ref, aggA_ref, aggB_ref, semb_ref, h0_ref, s_ref, xA_ref, xB_ref,
                W1At, W1Ab, b1A, W1Bt, W1Bb, b1B,
                Wf1_0, bf1_0, Wf2_0, bf2_0,
                Wf1_1, bf1_1, Wf2_1, bf2_1,
                Wf1_2, bf1_2, Wf2_2, bf2_2,
                Wms0, Wmd0, bm0,
                f0_ref, f1_ref, f2_ref, x_ref, ts_ref, td_ref):
    aggA = aggA_ref[0, :, :64] + aggA_ref[1, :, :64]
    aggB = aggB_ref[0, :, :64] + aggB_ref[1, :, :64]
    tA = tab_ref[:, :64]
    tB = tab_ref[:, 64:]
    hA = _gelu(_mm(tA, W1At[...]) + _mm(aggA, W1Ab[...]) + b1A[...])
    hB = _gelu(_mm(tB, W1Bt[...]) + _mm(aggB, W1Bb[...]) + b1B[...])
    fi = jnp.concatenate([hA, hB, semb_ref[...]], axis=1)
    f0_ref[...] = _mm(_gelu(_mm(fi, Wf1_0[...]) + bf1_0[...]), Wf2_0[...]) + bf2_0[...]
    f1_ref[...] = _mm(_gelu(_mm(fi, Wf1_1[...]) + bf1_1[...]), Wf2_1[...]) + bf2_1[...]
    f2_ref[...] = _mm(_gelu(_mm(fi, Wf1_2[...]) + bf1_2[...]), Wf2_2[...]) + bf2_2[...]
    h0 = h0_ref[...]
    sv = s_ref[...]
    x16 = (1.0 - sv) * xA_ref[...] + sv * xB_ref[...]
    x_ref[...] = x16
    xa4 = xA_ref[:, :4]
    xb4 = xB_ref[:, :4]
    x4 = x16[:, :4]
    padc = jnp.zeros((x4.shape[0], 52), jnp.float32)
    ts_ref[...] = jnp.concatenate([_mm(h0, Wms0[...]), x4, xa4, xb4, padc], axis=1)
    td_ref[...] = jnp.concatenate([_mm(h0, Wmd0[...]) + bm0[...], x4, xa4, xb4, padc], axis=1)


def _edge_dyn(dx4, ef, We_s, We_d):
    dist = jnp.sqrt(jnp.sum(dx4 * dx4, axis=1, keepdims=True) + 1e-12)
    efd = jnp.concatenate([dist, dist * dist, _rbf(dist)], axis=1)
    return _mm(ef, We_s) + _mm(efd, We_d)


def _msg0_body(gs_ref, gd_ref, bond_ref,
               We_s, We_d, Wm1, bm1, Wm2, bm2,
               msg_ref, dx_ref, ef_ref):
    gs = gs_ref[...]
    gd = gd_ref[...]
    dA4 = gs[:, 68:72] - gd[:, 68:72]
    dB4 = gs[:, 72:76] - gd[:, 72:76]
    dA = jnp.sqrt(jnp.sum(dA4 * dA4, axis=1, keepdims=True) + 1e-12)
    dB = jnp.sqrt(jnp.sum(dB4 * dB4, axis=1, keepdims=True) + 1e-12)
    zp = jnp.zeros((gs.shape[0], 7), jnp.float32)
    ef = jnp.concatenate([bond_ref[...], dA, dB, dA - dB, _rbf(dA), _rbf(dB), zp], axis=1)
    ef_ref[...] = ef
    dx4 = gs[:, 64:68] - gd[:, 64:68]
    pre = gs[:, :64] + gd[:, :64] + _edge_dyn(dx4, ef, We_s[...], We_d[...])
    z = _gelu(pre)
    z = _gelu(_mm(z, Wm1[...]) + bm1[...])
    msgv = _mm(z, Wm2[...]) + bm2[...]
    msg_ref[...] = jnp.concatenate(
        [msgv, jnp.zeros((msgv.shape[0], 64), jnp.float32)], axis=1).reshape(msg_ref.shape)
    dx_ref[...] = jnp.concatenate([dx4, jnp.zeros((gs.shape[0], 12), jnp.float32)], axis=1)


def _msg_body(gs_ref, gd_ref, ef_ref,
              We_s, We_d, Wm1, bm1, Wm2, bm2,
              msg_ref, dx_ref):
    gs = gs_ref[...]
    gd = gd_ref[...]
    dx4 = gs[:, 64:68] - gd[:, 64:68]
    pre = gs[:, :64] + gd[:, :64] + _edge_dyn(dx4, ef_ref[...], We_s[...], We_d[...])
    z = _gelu(pre)
    z = _gelu(_mm(z, Wm1[...]) + bm1[...])
    msgv = _mm(z, Wm2[...]) + bm2[...]
    msg_ref[...] = jnp.concatenate(
        [msgv, jnp.zeros((msgv.shape[0], 64), jnp.float32)], axis=1).reshape(msg_ref.shape)
    dx_ref[...] = jnp.concatenate([dx4, jnp.zeros((gs.shape[0], 12), jnp.float32)], axis=1)


def _state_body(h_ref, ms_ref, f_ref,
                Ws_h, Ws_m, bs1, Ws2, bs2, Ws3, bs3,
                Wa_s, Wa_d, ba,
                Wb1, bb1, Wb2, bb2, Wb3, bb3,
                Wg1, bg1, Wg2, bg2, Wg3, bg3,
                Wms, Wmd, bm,
                h_out, ta_ref, bg_ref, ums_ref, umd_ref):
    h = h_ref[...]
    nm = ms_ref[0, :, :64] + ms_ref[1, :, :64]
    u = _gelu(_mm(h, Ws_h[...]) + _mm(nm, Ws_m[...]) + bs1[...])
    u = _gelu(_mm(u, Ws2[...]) + bs2[...])
    h2 = h + _mm(u, Ws3[...]) + bs3[...]
    g = f_ref[:, :_STATE]
    b = f_ref[:, _STATE:]
    h2 = _gelu((1.0 + g) * h2 + b)
    h_out[...] = h2
    ta_ref[...] = jnp.concatenate([_mm(h2, Wa_s[...]), _mm(h2, Wa_d[...]) + ba[...]], axis=1)
    zb = _gelu(_mm(h2, Wb1[...]) + bb1[...])
    zb = _gelu(_mm(zb, Wb2[...]) + bb2[...])
    bet = _mm(zb, Wb3[...]) + bb3[...]
    zg = _gelu(_mm(h2, Wg1[...]) + bg1[...])
    zg = _gelu(_mm(zg, Wg2[...]) + bg2[...])
    gam = _mm(zg, Wg3[...]) + bg3[...]
    bg_ref[...] = jnp.concatenate([bet, gam], axis=1)
    ums_ref[...] = _mm(h2, Wms[...])
    umd_ref[...] = _mm(h2, Wmd[...]) + bm[...]


def _alpha_body(gs_ref, gd_ref, dx_ref, ef_ref,
                Wae_s, Wae_d, Wa1, ba1, Wa2, ba2,
                av_ref):
    dx16 = dx_ref[...]
    pre = (gs_ref[:, :64] + gd_ref[:, 64:]
           + _edge_dyn(dx16[:, :4], ef_ref[...], Wae_s[...], Wae_d[...]))
    z = _gelu(pre)
    z = _gelu(_mm(z, Wa1[...]) + ba1[...])
    alpha = _mm(z, Wa2[...]) + ba2[...]
    av_ref[...] = jnp.concatenate(
        [alpha * dx16, jnp.zeros((dx16.shape[0], 112), jnp.float32)],
        axis=1).reshape(av_ref.shape)


def _xup_body(x_ref, nu_ref, bg_ref, s_ref, xA_ref, xB_ref, ums_ref, umd_ref,
              xo_ref, ts_ref, td_ref):
    x = x_ref[...]
    nu = nu_ref[0, :, :16] + nu_ref[1, :, :16]
    bet = bg_ref[:, 0:1]
    gam = bg_ref[:, 1:2]
    sv = s_ref[...]
    xA = xA_ref[...]
    xB = xB_ref[...]
    xn = x + nu + bet * (1.0 - sv) * (xA - x) + gam * sv * (xB - x)
    xo_ref[...] = xn
    x4 = xn[:, :4]
    xa4 = xA[:, :4]
    xb4 = xB[:, :4]
    padc = jnp.zeros((x4.shape[0], 52), jnp.float32)
    ts_ref[...] = jnp.concatenate([ums_ref[...], x4, xa4, xb4, padc], axis=1)
    td_ref[...] = jnp.concatenate([umd_ref[...], x4, xa4, xb4, padc], axis=1)


def _final_body(x_ref, xA_ref, xB_ref, s_ref, out_ref):
    sv = s_ref[...]
    base = (1.0 - sv) * xA_ref[...] + sv * xB_ref[...]
    corr = x_ref[...] - base
    xf = base + sv * (1.0 - sv) * corr
    out_ref[...] = xf - jnp.sum(xf, axis=0, keepdims=True) * (1.0 / _N)


_NC = 2
_NS = 16
_NW = _NC * _NS
_SC_C = 80
_SC_K = 8
_SC_H = 4
_SC_G = _SC_C * _SC_K
_SC_B = _SC_C * _SC_H
_NGRP_TOT = _E // _SC_G
_GRP_BASE = _NGRP_TOT // _NW
_GRP_EXTRA = _NGRP_TOT - _GRP_BASE * _NW
_RPT = 624
_ZR = 16


@functools.cache
def _sc_mesh():
    return plsc.VectorSubcoreMesh(core_axis_name="c", subcore_axis_name="s")


def _sc_wid():
    return lax.axis_index("s") * _NC + lax.axis_index("c")


def _sc_grp_range(wid):
    g0 = wid * _GRP_BASE + jnp.minimum(wid, _GRP_EXTRA)
    ng = _GRP_BASE + (wid < _GRP_EXTRA).astype(jnp.int32)
    return g0, ng


def _sc_gather_multi(tables, idx3ds):
    K = len(tables)
    widths = [int(t.shape[1]) for t in tables]
    Q = 4
    R = _SC_C * 2
    out_type = [jax.ShapeDtypeStruct((_E, w), jnp.float32) for w in widths]
    scratch = ([pltpu.VMEM((_SC_K, _SC_C), jnp.int32) for _ in range(K)]
               + [pltpu.VMEM((2 * R, w), jnp.float32) for w in widths]
               + [pltpu.SemaphoreType.DMA for _ in range(K)]
               + [pltpu.SemaphoreType.DMA for _ in range(2 * K)])

    @functools.partial(pl.kernel, out_type=out_type, mesh=_sc_mesh(),
                       scratch_types=scratch)
    def run(*refs):
        t_refs = refs[:K]
        i_refs = refs[K:2 * K]
        o_refs = refs[2 * K:3 * K]
        idxb = refs[3 * K:4 * K]
        rowb = refs[4 * K:5 * K]
        gsems = refs[5 * K:6 * K]
        osems = refs[6 * K:8 * K]
        g0, ng = _sc_grp_range(_sc_wid())

        def group(g, _):
            grp = g0 + g
            for k in range(K):
                pltpu.sync_copy(i_refs[k].at[grp], idxb[k])
            for q in range(Q + 1):
                if q < Q:
                    p = q % 2
                    for k in range(K):
                        buf = rowb[k].at[pl.ds(p * R, R)]
                        osem = osems[2 * k + p]
                        @pl.when((g > 0) | (q >= 2))
                        def _():
                            pltpu.make_async_copy(
                                buf, o_refs[k].at[pl.ds(0, R)], osem).wait()
                        for j in range(2):
                            pltpu.async_copy(
                                t_refs[k].at[idxb[k].at[2 * q + j]],
                                buf.at[pl.ds(j * _SC_C, _SC_C)], gsems[k])
                if q >= 1:
                    pp = (q - 1) % 2
                    for k in range(K):
                        buf = rowb[k].at[pl.ds(pp * R, R)]
                        for j in range(2):
                            pltpu.make_async_copy(
                                t_refs[k].at[idxb[k].at[2 * (q - 1) + j]],
                                buf.at[pl.ds(j * _SC_C, _SC_C)],
                                gsems[k]).wait()
                        pltpu.async_copy(
                            buf,
                            o_refs[k].at[pl.ds(grp * _SC_G + (q - 1) * R, R)],
                            osems[2 * k + pp])
            return ()

        lax.fori_loop(0, ng, group, (), unroll=False)
        for k in range(K):
            for p in range(2):
                pltpu.make_async_copy(
                    rowb[k].at[pl.ds(p * R, R)],
                    o_refs[k].at[pl.ds(0, R)], osems[2 * k + p]).wait()

    return run(*tables, *idx3ds)


def _sc_zero_accum(z_ref, zbuf, accum, sid):
    pltpu.sync_copy(z_ref, zbuf)

    def zcp(j, _):
        pltpu.sync_copy(zbuf, accum.at[pl.ds(sid * _RPT + j * _ZR, _ZR)])
        return ()

    lax.fori_loop(0, _RPT // _ZR, zcp, (), unroll=False)

    @pl.when(sid == _NS - 1)
    def _():
        pltpu.sync_copy(zbuf, accum.at[pl.ds(_NS * _RPT, _ZR)])


def _sc_accum_out(accum, o_ref, cid, sid):
    @pl.when(sid < _NS - 1)
    def _():
        pltpu.sync_copy(accum.at[pl.ds(sid * _RPT, _RPT)],
                        o_ref.at[cid, pl.ds(sid * _RPT, _RPT)])

    @pl.when(sid == _NS - 1)
    def _():
        pltpu.sync_copy(accum.at[pl.ds((_NS - 1) * _RPT, _RPT + _ZR)],
                        o_ref.at[cid, pl.ds((_NS - 1) * _RPT, _RPT + _ZR)])


def _sc_scatter_add2(data, idx3d):
    w = 128
    out_type = jax.ShapeDtypeStruct((_NC, _N, w), jnp.float32)
    scratch = [pltpu.VMEM((_SC_K, _SC_C), jnp.int32),
               pltpu.VMEM((_SC_B, 128), jnp.float32),
               pltpu.VMEM((_ZR, w), jnp.float32),
               pltpu.VMEM_SHARED((_N, w), jnp.float32),
               pltpu.SemaphoreType.DMA]

    @functools.partial(pl.kernel, out_type=out_type, mesh=_sc_mesh(),
                       scratch_types=scratch)
    def run(d_ref, i_ref, z_ref, o_ref, idxb, rowb, zbuf, accum, lsem):
        cid = lax.axis_index("c")
        sid = lax.axis_index("s")
        g0, ng = _sc_grp_range(_sc_wid())
        _sc_zero_accum(z_ref, zbuf, accum, sid)
        plsc.subcore_barrier()
        H = _SC_B // 2

        def group(g, _):
            grp = g0 + g
            pltpu.sync_copy(i_ref.at[grp], idxb)
            pltpu.async_copy(d_ref.at[2 * grp].at[pl.ds(0, H)],
                             rowb.at[pl.ds(0, H)], lsem)
            for q in range(4):
                p = q % 2
                buf = rowb.at[pl.ds(p * H, H)]
                pltpu.make_async_copy(d_ref.at[2 * grp].at[pl.ds(0, H)],
                                      buf, lsem).wait()
                if q < 3:
                    qs = q + 1
                    pltpu.async_copy(
                        d_ref.at[2 * grp + qs // 2].at[pl.ds((qs % 2) * H, H)],
                        rowb.at[pl.ds((1 - p) * H, H)], lsem)
                for j in range(2):
                    pltpu.sync_copy(buf.at[pl.ds(j * _SC_C, _SC_C)],
                                    accum.at[idxb.at[2 * q + j]], add=True)
            return ()

        lax.fori_loop(0, ng, group, (), unroll=False)
        plsc.subcore_barrier()
        _sc_accum_out(accum, o_ref, cid, sid)

    return run(data, idx3d, jnp.zeros((_ZR, w), jnp.float32))


def _pack_edge_first(W, b):
    Ws = W[:_STATE]
    Wd = W[_STATE:2 * _STATE]
    We = W[2 * _STATE:]
    stat = jnp.concatenate([We[0][None], We[1][None], We[4][None], We[5][None],
                            We[6][None], We[17:27], We[27:37],
                            jnp.zeros((7, We.shape[1]), jnp.float32)], axis=0)
    dyn = jnp.concatenate([We[2][None], We[3][None], We[7:17]], axis=0)
    return Ws, Wd, stat, dyn, b[None, :]


def _r2(b):
    return b[None, :]


def kernel(xA_pos, xB_pos, s, is_bond_A, is_bond_B, params, Z, edge_index):
    s2 = s.reshape(_N, 1)
    z2 = Z.reshape(_N, 1).astype(jnp.int32)
    src3d = edge_index[0].astype(jnp.int32).reshape(_E // _SC_G, _SC_K, _SC_C)
    dst3d = edge_index[1].astype(jnp.int32).reshape(_E // _SC_G, _SC_K, _SC_C)
    pad13 = jnp.zeros((_N, 13), jnp.float32)
    xA16 = jnp.concatenate([xA_pos, pad13], axis=1)
    xB16 = jnp.concatenate([xB_pos, pad13], axis=1)
    bond2 = jnp.stack([is_bond_A, is_bond_B], axis=1)

    p = params
    Wi0, bi0 = p['info'][0]
    Wi1, bi1 = p['info'][1]
    WA0, bA0 = p['embA'][0]
    WB0, bB0 = p['embB'][0]
    W1A, b1A = p['embA'][1]
    W1B, b1B = p['embB'][1]

    tab, h0, semb = _rowcall(
        _node0_body, _N, _NBLK, [z2, s2],
        [Wi0, _r2(bi0), Wi1, _r2(bi1), WA0, _r2(bA0), WB0, _r2(bB0)],
        [128, 80, 16])

    tsrc = _sc_gather_multi([tab], [src3d])[0]
    eA, eB = _rowcall(_split_body, _E, _EBLK, [tsrc], [], [('e3', 128), ('e3', 128)])
    aggA = _sc_scatter_add2(eA, dst3d)
    aggB = _sc_scatter_add2(eB, dst3d)

    lw = p['layers']
    film_w = []
    for l in range(_N_LAYERS):
        Wf1, bf1 = lw[l]['film'][0]
        Wf2, bf2 = lw[l]['film'][1]
        film_w += [Wf1, _r2(bf1), Wf2, _r2(bf2)]
    msg_first = [_pack_edge_first(*lw[l]['msg'][0]) for l in range(_N_LAYERS)]
    alpha_first = [_pack_edge_first(*lw[l]['alpha'][0]) for l in range(_N_LAYERS)]

    f0, f1, f2, x16, ts, td = _rowcall(
        _node1_body, _N, _NBLK, [tab, aggA, aggB, semb, h0, s2, xA16, xB16],
        [W1A[:64], W1A[64:], _r2(b1A), W1B[:64], W1B[64:], _r2(b1B)]
        + film_w
        + [msg_first[0][0], msg_first[0][1], msg_first[0][4]],
        [160, 160, 160, 16, 128, 128])
    films = [f0, f1, f2]

    h = h0
    ef = None
    dx16 = None
    for l in range(_N_LAYERS):
        lp = lw[l]
        _, _, We_s, We_d, _ = msg_first[l]
        Wa_s, Wa_d, Wae_s, Wae_d, ba = alpha_first[l]

        gs, gd = _sc_gather_multi([ts, td], [src3d, dst3d])
        if l == 0:
            msg, dx16, ef = _rowcall(
                _msg0_body, _E, _EBLK, [gs, gd, bond2],
                [We_s, We_d, lp['msg'][1][0], _r2(lp['msg'][1][1]),
                 lp['msg'][2][0], _r2(lp['msg'][2][1])],
                [('e3', 128), 16, 32])
        else:
            msg, dx16 = _rowcall(
                _msg_body, _E, _EBLK, [gs, gd, ef],
                [We_s, We_d, lp['msg'][1][0], _r2(lp['msg'][1][1]),
                 lp['msg'][2][0], _r2(lp['msg'][2][1])],
                [('e3', 128), 16])

        mslab = _sc_scatter_add2(msg, dst3d)

        Ws0, bs0 = lp['state'][0]
        next_l = min(l + 1, _N_LAYERS - 1)
        Wms_n, Wmd_n, _, _, bm_n = msg_first[next_l]
        h, ta, bgv, ums, umd = _rowcall(
            _state_body, _N, _NBLK, [h, mslab, films[l]],
            [Ws0[:_STATE], Ws0[_STATE:], _r2(bs0),
             lp['state'][1][0], _r2(lp['state'][1][1]),
             lp['state'][2][0], _r2(lp['state'][2][1]),
             Wa_s, Wa_d, ba,
             lp['beta'][0][0], _r2(lp['beta'][0][1]),
             lp['beta'][1][0], _r2(lp['beta'][1][1]),
             lp['beta'][2][0], _r2(lp['beta'][2][1]),
             lp['gamma'][0][0], _r2(lp['gamma'][0][1]),
             lp['gamma'][1][0], _r2(lp['gamma'][1][1]),
             lp['gamma'][2][0], _r2(lp['gamma'][2][1]),
             Wms_n, Wmd_n, bm_n],
            [80, 128, 2, 64, 64])

        gas, gad = _sc_gather_multi([ta, ta], [src3d, dst3d])
        avdx = _rowcall(
            _alpha_body, _E, _EBLK, [gas, gad, dx16, ef],
            [Wae_s, Wae_d, lp['alpha'][1][0], _r2(lp['alpha'][1][1]),
             lp['alpha'][2][0], _r2(lp['alpha'][2][1])],
            [('e3', 128)])[0]

        nuslab = _sc_scatter_add2(avdx, dst3d)

        x16, ts, td = _rowcall(
            _xup_body, _N, _NBLK, [x16, nuslab, bgv, s2, xA16, xB16, ums, umd],
            [], [16, 128, 128])

    out = pl.pallas_call(
        _final_body,
        grid=(1,),
        in_specs=[pl.BlockSpec((_N, 16), lambda i: (0, 0))] * 3
        + [pl.BlockSpec((_N, 1), lambda i: (0, 0))],
        out_specs=pl.BlockSpec((_N, 16), lambda i: (0, 0)),
        out_shape=jax.ShapeDtypeStruct((_N, 16), jnp.float32),
    )(x16, xA16, xB16, s2)
    return out[:, :3]
